# baseline (speedup 1.0000x reference)
import jax
import jax.numpy as jnp
from jax.experimental import pallas as pl


def _probe_kernel(x_ref, o_ref):
    o_ref[...] = jnp.sum(x_ref[0], axis=0)


def _pallas_probe(x4):
    N, C, H, Wd = x4.shape
    x = x4.reshape(N, C, 8, (H * Wd) // 8)
    return pl.pallas_call(
        _probe_kernel,
        grid=(N,),
        in_specs=[pl.BlockSpec((1, C, 8, (H * Wd) // 8), lambda i: (i, 0, 0, 0))],
        out_specs=pl.BlockSpec((8, (H * Wd) // 8), lambda i: (0, 0)),
        out_shape=jax.ShapeDtypeStruct((8, (H * Wd) // 8), jnp.float32),
    )(x)


def _dump():
    try:
        f = jnp.zeros((4, 768, 32, 32), jnp.float32)
        txt = jax.jit(_pallas_probe).lower(f).compile().as_text()
        lines = [l for l in txt.splitlines()
                 if ("fusion" in l or "custom-call" in l or "copy" in l
                     or "bitcast" in l or "ENTRY" in l or "parameter" in l
                     or "reshape" in l or "transpose" in l)]
        print("==== PROBE-E pallas HLO (filtered) ====")
        for l in lines[:60]:
            print(l[:400])
        g = jax.jit(lambda f: jnp.mean(f, axis=(2, 3))).lower(f).compile().as_text()
        print("==== PROBE-E xla reduce HLO (filtered) ====")
        for l in g.splitlines():
            if ("fusion" in l or "ENTRY" in l or "parameter" in l
                    or "reduce" in l or "copy" in l):
                print(l[:400])
    except Exception as e:
        print("PROBE-E dump failed:", repr(e))


_dump()


def kernel(features, depth, W, b):
    return _pallas_probe(features)


# speedup vs baseline: 2.1810x; 2.1810x over previous
import jax
import jax.numpy as jnp
from jax.experimental import pallas as pl


def _pool_conv_broadcast_kernel(x_ref, wt_ref, b_ref, o_ref):
    x = x_ref[0]
    m = jnp.sum(x, axis=0, keepdims=True) * (1.0 / x.shape[0])
    pooled = jnp.dot(m, wt_ref[...],
                     preferred_element_type=jnp.float32) + b_ref[...]
    o_ref[0] = jnp.broadcast_to(pooled, o_ref.shape[1:])


def kernel(features, depth, W, b):
    del depth
    N, C, H, Wd = features.shape
    O = W.shape[0]
    HW = H * Wd
    x = features.transpose(0, 2, 3, 1).reshape(N, HW, C)
    wt = W.reshape(O, C).T
    b2 = b.reshape(1, O)
    out = pl.pallas_call(
        _pool_conv_broadcast_kernel,
        grid=(N,),
        in_specs=[
            pl.BlockSpec((1, HW, C), lambda i: (i, 0, 0)),
            pl.BlockSpec((C, O), lambda i: (0, 0)),
            pl.BlockSpec((1, O), lambda i: (0, 0)),
        ],
        out_specs=pl.BlockSpec((1, HW, O), lambda i: (i, 0, 0)),
        out_shape=jax.ShapeDtypeStruct((N, HW, O), jnp.float32),
    )(x, wt, b2)
    return out.reshape(N, H, Wd, O).transpose(0, 3, 1, 2)


# speedup vs baseline: 2.2080x; 1.0124x over previous
import jax
import jax.numpy as jnp
from jax.experimental import pallas as pl

_S = 4


def _pool_conv_broadcast_kernel(*refs):
    xs = refs[:_S]
    wt_ref, b_ref, o_ref = refs[_S], refs[_S + 1], refs[_S + 2]
    hw = o_ref.shape[1]
    m = xs[0][0, 0].sum(axis=0, keepdims=True)
    for x in xs[1:]:
        m = m + x[0, 0].sum(axis=0, keepdims=True)
    pooled = jnp.dot(m * (1.0 / hw), wt_ref[...],
                     preferred_element_type=jnp.float32) + b_ref[...]
    o_ref[0] = jnp.broadcast_to(pooled, o_ref.shape[1:])


def kernel(features, depth, W, b):
    del depth
    N, C, H, Wd = features.shape
    O = W.shape[0]
    HW = H * Wd
    R = HW // _S
    x = features.transpose(0, 2, 3, 1).reshape(N, _S, R, C)
    wt = W.reshape(O, C).T
    b2 = b.reshape(1, O)
    x_specs = [
        pl.BlockSpec((1, 1, R, C), lambda i, s=s: (i, s, 0, 0)) for s in range(_S)
    ]
    out = pl.pallas_call(
        _pool_conv_broadcast_kernel,
        grid=(N,),
        in_specs=x_specs + [
            pl.BlockSpec((C, O), lambda i: (0, 0)),
            pl.BlockSpec((1, O), lambda i: (0, 0)),
        ],
        out_specs=pl.BlockSpec((1, HW, O), lambda i: (i, 0, 0)),
        out_shape=jax.ShapeDtypeStruct((N, HW, O), jnp.float32),
    )(*([x] * _S), wt, b2)
    return out.reshape(N, H, Wd, O).transpose(0, 3, 1, 2)
